# baseline (speedup 1.0000x reference)
import functools

import jax
import jax.numpy as jnp
from jax import lax
from jax.experimental import pallas as pl
from jax.experimental.pallas import tpu as pltpu
from jax.experimental.pallas import tpu_sc as plsc

N_NODES = 10000
N_EDGES = 320000
D_FEAT = 128
D_EDGE = 16

NC = 2
NS = 16
NW = NC * NS
EPW = N_EDGES // NW
CH = 80
NCH = EPW // CH
NPAD = 10112
RPT = NPAD // NS


def _sc_phase_a(x, src2, dst2, zx):
    mesh = plsc.VectorSubcoreMesh(core_axis_name="c", subcore_axis_name="s")

    @functools.partial(
        pl.kernel,
        out_type=jax.ShapeDtypeStruct((NC, NPAD, D_FEAT), jnp.float32),
        mesh=mesh,
        scratch_types=[
            pltpu.VMEM((EPW,), jnp.int32),
            pltpu.VMEM((NCH, CH), jnp.int32),
            pltpu.VMEM((2, CH, D_FEAT), jnp.float32),
            pltpu.VMEM_SHARED((NPAD, D_FEAT), jnp.float32),
            pltpu.SemaphoreType.DMA((2,)),
            pltpu.SemaphoreType.DMA((2,)),
        ],
    )
    def k(x_hbm, src_hbm, dst_hbm, zx_hbm, gx_out,
          src_v, dst_v, xrows_v, acc_sh, gsem, ssem):
        c = lax.axis_index("c")
        s = lax.axis_index("s")
        wid = c * NS + s

        pltpu.sync_copy(zx_hbm, acc_sh.at[pl.ds(s * RPT, RPT)])
        pltpu.sync_copy(src_hbm.at[wid], src_v)
        pltpu.sync_copy(dst_hbm.at[wid], dst_v)
        plsc.subcore_barrier()

        def gather(buf, i):
            pltpu.async_copy(x_hbm.at[src_v.at[pl.ds(i * CH, CH)]],
                             xrows_v.at[buf], gsem.at[buf])

        def wait_gather(buf):
            pltpu.make_async_copy(x_hbm.at[src_v.at[pl.ds(0, CH)]],
                                  xrows_v.at[buf], gsem.at[buf]).wait()

        def wait_scat(buf):
            pltpu.make_async_copy(xrows_v.at[buf], acc_sh.at[dst_v.at[0]],
                                  ssem.at[buf]).wait()

        pltpu.sync_copy(zx_hbm.at[pl.ds(0, CH)], xrows_v.at[1])
        pltpu.async_copy(xrows_v.at[1], acc_sh.at[dst_v.at[0]], ssem.at[1],
                         add=True)
        gather(0, 0)

        def step(i, b, bo):
            wait_scat(bo)
            gather(bo, i + 1)
            wait_gather(b)
            pltpu.async_copy(xrows_v.at[b], acc_sh.at[dst_v.at[i]],
                             ssem.at[b], add=True)

        def pair(g, carry):
            step(2 * g, 0, 1)
            step(2 * g + 1, 1, 0)
            return carry

        lax.fori_loop(0, (NCH - 1) // 2, pair, 0)
        wait_gather(0)
        pltpu.async_copy(xrows_v.at[0], acc_sh.at[dst_v.at[NCH - 1]],
                         ssem.at[0], add=True)
        wait_scat(1)
        wait_scat(0)
        plsc.subcore_barrier()
        pltpu.sync_copy(acc_sh.at[pl.ds(s * RPT, RPT)],
                        gx_out.at[c, pl.ds(s * RPT, RPT)])

    return k(x, src2, dst2, zx)


EW = CH * D_EDGE // D_FEAT


def _sc_phase_b(dst2, ea4, zx, tmpl):
    mesh = plsc.VectorSubcoreMesh(core_axis_name="c", subcore_axis_name="s")

    @functools.partial(
        pl.kernel,
        out_type=jax.ShapeDtypeStruct((NC, NPAD, D_FEAT), jnp.float32),
        mesh=mesh,
        scratch_types=[
            pltpu.VMEM((NCH, CH), jnp.int32),
            pltpu.VMEM((2, EW, D_FEAT), jnp.float32),
            pltpu.VMEM((2, CH, D_FEAT), jnp.float32),
            pltpu.VMEM_SHARED((NPAD, D_FEAT), jnp.float32),
            pltpu.SemaphoreType.DMA((2,)),
            pltpu.SemaphoreType.DMA((2,)),
        ],
    )
    def k(dst_hbm, ea_hbm, zx_hbm, tm_hbm, ge_out,
          dst_v, ew_v, asm_v, acc_sh, gsem, ssem):
        c = lax.axis_index("c")
        s = lax.axis_index("s")
        wid = c * NS + s

        pltpu.sync_copy(zx_hbm, acc_sh.at[pl.ds(s * RPT, RPT)])
        pltpu.sync_copy(dst_hbm.at[wid], dst_v)
        pltpu.sync_copy(tm_hbm, asm_v.at[0])
        pltpu.sync_copy(tm_hbm, asm_v.at[1])
        plsc.subcore_barrier()

        def eload(buf, i):
            pltpu.async_copy(ea_hbm.at[wid, i], ew_v.at[buf], gsem.at[buf])

        def wait_eload(buf):
            pltpu.make_async_copy(ea_hbm.at[0, 0], ew_v.at[buf],
                                  gsem.at[buf]).wait()

        def wait_scat(buf):
            pltpu.make_async_copy(asm_v.at[buf], acc_sh.at[dst_v.at[0]],
                                  ssem.at[buf]).wait()

        def assemble(b):
            for r in range(CH):
                asm_v[b, r, pl.ds(0, D_EDGE)] = ew_v[b, r // 8,
                                                     pl.ds((r % 8) * D_EDGE,
                                                           D_EDGE)]

        def scat(b, i):
            pltpu.async_copy(asm_v.at[b], acc_sh.at[dst_v.at[i]],
                             ssem.at[b], add=True)

        pltpu.sync_copy(ea_hbm.at[wid, 0], ew_v.at[0])
        eload(1, 1)
        assemble(0)
        scat(0, 0)

        def step(i, b, bo):
            wait_scat(bo)
            eload(bo, i + 1)
            wait_eload(b)
            assemble(b)
            scat(b, i)

        def pair(g, carry):
            step(2 * g + 1, 1, 0)
            step(2 * g + 2, 0, 1)
            return carry

        lax.fori_loop(0, (NCH - 3) // 2, pair, 0)
        step(NCH - 2, 1, 0)
        wait_eload(0)
        assemble(0)
        scat(0, NCH - 1)
        wait_scat(1)
        wait_scat(0)
        plsc.subcore_barrier()
        pltpu.sync_copy(acc_sh.at[pl.ds(s * RPT, RPT)],
                        ge_out.at[c, pl.ds(s * RPT, RPT)])

    return k(dst2, ea4, zx, tmpl)


def _tc_epilogue_body(x_ref, gx_ref, ge_ref,
                      ws_ref, we_ref, wf_ref, b_ref, o_ref):
    gx = gx_ref[0] + gx_ref[1]
    geo = ge_ref[0] + ge_ref[1]
    ge = geo[:, 0:D_EDGE]
    deg = geo[:, D_EDGE:D_EDGE + 1]
    agg = (jnp.dot(gx, ws_ref[...], preferred_element_type=jnp.float32)
           + jnp.dot(ge, we_ref[...], preferred_element_type=jnp.float32))
    agg = agg / jnp.maximum(deg, 1.0)
    self_t = jnp.dot(x_ref[...], wf_ref[...], preferred_element_type=jnp.float32)
    o_ref[...] = jnp.maximum(self_t + agg + b_ref[...], 0.0)


def _tc_epilogue(x, gx2, ge2, W_src, W_edge, W_self, b):
    BR = 2000
    grid = (N_NODES // BR,)
    return pl.pallas_call(
        _tc_epilogue_body,
        grid=grid,
        in_specs=[
            pl.BlockSpec((BR, D_FEAT), lambda i: (i, 0)),
            pl.BlockSpec((NC, BR, D_FEAT), lambda i: (0, i, 0)),
            pl.BlockSpec((NC, BR, D_FEAT), lambda i: (0, i, 0)),
            pl.BlockSpec((D_FEAT, D_FEAT), lambda i: (0, 0)),
            pl.BlockSpec((D_EDGE, D_FEAT), lambda i: (0, 0)),
            pl.BlockSpec((D_FEAT, D_FEAT), lambda i: (0, 0)),
            pl.BlockSpec((1, D_FEAT), lambda i: (0, 0)),
        ],
        out_specs=pl.BlockSpec((BR, D_FEAT), lambda i: (i, 0)),
        out_shape=jax.ShapeDtypeStruct((N_NODES, D_FEAT), jnp.float32),
    )(x, gx2, ge2, W_src, W_edge, W_self, b)


def kernel(x, edge_index, edge_attr, W_src, W_edge, W_self, b):
    src2 = edge_index[0].reshape(NW, EPW)
    dst2 = edge_index[1].reshape(NW, NCH, CH)
    zx = jnp.zeros((RPT, D_FEAT), jnp.float32)
    tmpl = jnp.concatenate(
        [jnp.zeros((CH, D_EDGE), jnp.float32),
         jnp.ones((CH, D_EDGE), jnp.float32),
         jnp.zeros((CH, D_FEAT - 2 * D_EDGE), jnp.float32)], axis=1)
    ea4 = edge_attr.reshape(NW, NCH, EW, D_FEAT)
    gx2 = _sc_phase_a(x, src2, dst2, zx)
    ge2 = _sc_phase_b(dst2, ea4, zx, tmpl)
    return _tc_epilogue(x, gx2, ge2, W_src, W_edge, W_self,
                        b.reshape(1, D_FEAT))


# speedup vs baseline: 1.0187x; 1.0187x over previous
import functools

import jax
import jax.numpy as jnp
from jax import lax
from jax.experimental import pallas as pl
from jax.experimental.pallas import tpu as pltpu
from jax.experimental.pallas import tpu_sc as plsc

N_NODES = 10000
N_EDGES = 320000
D_FEAT = 128
D_EDGE = 16

NC = 2
NS = 16
NW = NC * NS
EPW = N_EDGES // NW
CH = 80
NCH = EPW // CH
NPAD = 10112
RPT = NPAD // NS


def _sc_phase_a(x, src2, dst2, zx):
    mesh = plsc.VectorSubcoreMesh(core_axis_name="c", subcore_axis_name="s")

    @functools.partial(
        pl.kernel,
        out_type=jax.ShapeDtypeStruct((NC, NPAD, D_FEAT), jnp.float32),
        mesh=mesh,
        scratch_types=[
            pltpu.VMEM((3, CH), jnp.int32),
            pltpu.VMEM((3, 1, CH), jnp.int32),
            pltpu.VMEM((3, CH, D_FEAT), jnp.float32),
            pltpu.VMEM_SHARED((NPAD, D_FEAT), jnp.float32),
            pltpu.SemaphoreType.DMA((3,)),
            pltpu.SemaphoreType.DMA((3,)),
            pltpu.SemaphoreType.DMA((3,)),
        ],
    )
    def k(x_hbm, src_hbm, dst_hbm, zx_hbm, gx_out,
          sidx_v, didx_v, xrows_v, acc_sh, isem, gsem, ssem):
        c = lax.axis_index("c")
        s = lax.axis_index("s")
        wid = c * NS + s

        pltpu.sync_copy(zx_hbm, acc_sh.at[pl.ds(s * RPT, RPT)])
        plsc.subcore_barrier()

        def idxload(buf, i):
            pltpu.async_copy(src_hbm.at[wid, i], sidx_v.at[buf], isem.at[buf])
            pltpu.async_copy(dst_hbm.at[wid, i], didx_v.at[buf, 0],
                             isem.at[buf])

        def wait_idx(buf):
            pltpu.make_async_copy(src_hbm.at[0, 0], sidx_v.at[buf],
                                  isem.at[buf]).wait()
            pltpu.make_async_copy(dst_hbm.at[0, 0], didx_v.at[buf, 0],
                                  isem.at[buf]).wait()

        def gather(buf, i):
            pltpu.async_copy(x_hbm.at[sidx_v.at[buf]],
                             xrows_v.at[buf], gsem.at[buf])

        def wait_gather(buf):
            pltpu.make_async_copy(x_hbm.at[sidx_v.at[0]],
                                  xrows_v.at[buf], gsem.at[buf]).wait()

        def scat(buf):
            pltpu.async_copy(xrows_v.at[buf], acc_sh.at[didx_v.at[buf, 0]],
                             ssem.at[buf], add=True)

        def wait_scat(buf):
            pltpu.make_async_copy(xrows_v.at[buf],
                                  acc_sh.at[didx_v.at[0, 0]],
                                  ssem.at[buf]).wait()

        idxload(0, 0)
        idxload(1, 1)
        wait_idx(0)
        gather(0, 0)

        def body(i, carry):
            b0 = lax.rem(i, 3)
            b1 = lax.rem(i + 1, 3)
            b2 = lax.rem(i + 2, 3)

            @pl.when(jnp.logical_and(i >= 1, i < NCH - 2))
            def _():
                wait_scat(b2)

            @pl.when(i < NCH - 2)
            def _():
                idxload(b2, i + 2)

            @pl.when(i < NCH - 1)
            def _():
                wait_idx(b1)
                gather(b1, i + 1)

            wait_gather(b0)
            scat(b0)
            return carry

        lax.fori_loop(0, NCH, body, 0)

        def drain(d, carry):
            wait_scat(d)
            return carry

        lax.fori_loop(0, 3, drain, 0)
        plsc.subcore_barrier()
        pltpu.sync_copy(acc_sh.at[pl.ds(s * RPT, RPT)],
                        gx_out.at[c, pl.ds(s * RPT, RPT)])

    return k(x, src2, dst2, zx)


EW = CH * D_EDGE // D_FEAT


def _sc_phase_b(dst2, ea4, zx, tmpl):
    mesh = plsc.VectorSubcoreMesh(core_axis_name="c", subcore_axis_name="s")

    @functools.partial(
        pl.kernel,
        out_type=jax.ShapeDtypeStruct((NC, NPAD, D_FEAT), jnp.float32),
        mesh=mesh,
        scratch_types=[
            pltpu.VMEM((NCH, CH), jnp.int32),
            pltpu.VMEM((2, EW, D_FEAT), jnp.float32),
            pltpu.VMEM((2, CH, D_FEAT), jnp.float32),
            pltpu.VMEM_SHARED((NPAD, D_FEAT), jnp.float32),
            pltpu.SemaphoreType.DMA((2,)),
            pltpu.SemaphoreType.DMA((2,)),
        ],
    )
    def k(dst_hbm, ea_hbm, zx_hbm, tm_hbm, ge_out,
          dst_v, ew_v, asm_v, acc_sh, gsem, ssem):
        c = lax.axis_index("c")
        s = lax.axis_index("s")
        wid = c * NS + s

        pltpu.sync_copy(zx_hbm, acc_sh.at[pl.ds(s * RPT, RPT)])
        pltpu.sync_copy(dst_hbm.at[wid], dst_v)
        pltpu.sync_copy(tm_hbm, asm_v.at[0])
        pltpu.sync_copy(tm_hbm, asm_v.at[1])
        plsc.subcore_barrier()

        def eload(buf, i):
            pltpu.async_copy(ea_hbm.at[wid, i], ew_v.at[buf], gsem.at[buf])

        def wait_eload(buf):
            pltpu.make_async_copy(ea_hbm.at[0, 0], ew_v.at[buf],
                                  gsem.at[buf]).wait()

        def wait_scat(buf):
            pltpu.make_async_copy(asm_v.at[buf], acc_sh.at[dst_v.at[0]],
                                  ssem.at[buf]).wait()

        def assemble(b):
            for r in range(CH):
                asm_v[b, r, pl.ds(0, D_EDGE)] = ew_v[b, r // 8,
                                                     pl.ds((r % 8) * D_EDGE,
                                                           D_EDGE)]

        def scat(b, i):
            pltpu.async_copy(asm_v.at[b], acc_sh.at[dst_v.at[i]],
                             ssem.at[b], add=True)

        pltpu.sync_copy(ea_hbm.at[wid, 0], ew_v.at[0])
        eload(1, 1)
        assemble(0)
        scat(0, 0)

        def step(i, b, bo):
            wait_scat(bo)
            eload(bo, i + 1)
            wait_eload(b)
            assemble(b)
            scat(b, i)

        def pair(g, carry):
            step(2 * g + 1, 1, 0)
            step(2 * g + 2, 0, 1)
            return carry

        lax.fori_loop(0, (NCH - 3) // 2, pair, 0)
        step(NCH - 2, 1, 0)
        wait_eload(0)
        assemble(0)
        scat(0, NCH - 1)
        wait_scat(1)
        wait_scat(0)
        plsc.subcore_barrier()
        pltpu.sync_copy(acc_sh.at[pl.ds(s * RPT, RPT)],
                        ge_out.at[c, pl.ds(s * RPT, RPT)])

    return k(dst2, ea4, zx, tmpl)


def _tc_epilogue_body(x_ref, gx_ref, ge_ref,
                      ws_ref, we_ref, wf_ref, b_ref, o_ref):
    gx = gx_ref[0] + gx_ref[1]
    geo = ge_ref[0] + ge_ref[1]
    ge = geo[:, 0:D_EDGE]
    deg = geo[:, D_EDGE:D_EDGE + 1]
    agg = (jnp.dot(gx, ws_ref[...], preferred_element_type=jnp.float32)
           + jnp.dot(ge, we_ref[...], preferred_element_type=jnp.float32))
    agg = agg / jnp.maximum(deg, 1.0)
    self_t = jnp.dot(x_ref[...], wf_ref[...], preferred_element_type=jnp.float32)
    o_ref[...] = jnp.maximum(self_t + agg + b_ref[...], 0.0)


def _tc_epilogue(x, gx2, ge2, W_src, W_edge, W_self, b):
    BR = 2000
    grid = (N_NODES // BR,)
    return pl.pallas_call(
        _tc_epilogue_body,
        grid=grid,
        in_specs=[
            pl.BlockSpec((BR, D_FEAT), lambda i: (i, 0)),
            pl.BlockSpec((NC, BR, D_FEAT), lambda i: (0, i, 0)),
            pl.BlockSpec((NC, BR, D_FEAT), lambda i: (0, i, 0)),
            pl.BlockSpec((D_FEAT, D_FEAT), lambda i: (0, 0)),
            pl.BlockSpec((D_EDGE, D_FEAT), lambda i: (0, 0)),
            pl.BlockSpec((D_FEAT, D_FEAT), lambda i: (0, 0)),
            pl.BlockSpec((1, D_FEAT), lambda i: (0, 0)),
        ],
        out_specs=pl.BlockSpec((BR, D_FEAT), lambda i: (i, 0)),
        out_shape=jax.ShapeDtypeStruct((N_NODES, D_FEAT), jnp.float32),
    )(x, gx2, ge2, W_src, W_edge, W_self, b)


def kernel(x, edge_index, edge_attr, W_src, W_edge, W_self, b):
    src2 = edge_index[0].reshape(NW, NCH, CH)
    dst2 = edge_index[1].reshape(NW, NCH, CH)
    zx = jnp.zeros((RPT, D_FEAT), jnp.float32)
    tmpl = jnp.concatenate(
        [jnp.zeros((CH, D_EDGE), jnp.float32),
         jnp.ones((CH, D_EDGE), jnp.float32),
         jnp.zeros((CH, D_FEAT - 2 * D_EDGE), jnp.float32)], axis=1)
    ea4 = edge_attr.reshape(NW, NCH, EW, D_FEAT)
    gx2 = _sc_phase_a(x, src2, dst2, zx)
    ge2 = _sc_phase_b(dst2, ea4, zx, tmpl)
    return _tc_epilogue(x, gx2, ge2, W_src, W_edge, W_self,
                        b.reshape(1, D_FEAT))
